# baseline (speedup 1.0000x reference)
import functools
import math

import jax
import jax.numpy as jnp
import numpy as np
from jax import lax
from jax._src import config as _jax_config
from jax.experimental import pallas as pl
from jax.experimental.pallas import tpu as pltpu
from jax.experimental.pallas import tpu_sc as plsc

N_MAX = 2048
N_MIN = 16
L = 16
T = 2 ** 19
F = 2
B = 131072
MASK = T - 1

PI1 = np.int32(-1640531535)
PI2 = np.int32(805459861)

NC = 2
NS = 16
NW = NC * NS
PW = B // NW
C = 256
NG = C // 16
NCHUNK = PW // C
GATHER_W = 128
NIDX = 16 * C
NGATHER = NIDX // GATHER_W
TSLICE = T * F // NS

_b = math.exp((math.log(N_MAX) - math.log(N_MIN)) / L)
SCALES = [np.float32(N_MIN * (_b ** i)) for i in range(L)]


def _encoder_body(xT_hbm, tbl_hbm, out_hbm,
                  xs, idx_v, feat_v, wbuf, obuf, spm, sem):
  sid = lax.axis_index("s")
  wid = sid * np.int32(NC) + lax.axis_index("c")
  pbase = wid * np.int32(PW)

  for d in range(3):
    pltpu.sync_copy(xT_hbm.at[pl.ds(np.int32(d * B) + pbase, PW)],
                    xs.at[pl.ds(d * PW, PW)])

  iota = lax.iota(jnp.int32, 16)

  for l in range(L):
    scale = SCALES[l]
    soff = sid * np.int32(TSLICE)
    pltpu.async_copy(
        tbl_hbm.at[pl.ds(np.int32(l * T * F) + soff, TSLICE)],
        spm.at[pl.ds(soff, TSLICE)], sem).wait()
    plsc.subcore_barrier()

    @pl.loop(np.int32(0), np.int32(NCHUNK), step=np.int32(1))
    def _chunk(chunk):
      coff = chunk * np.int32(C)

      @pl.loop(np.int32(0), np.int32(NG), step=np.int32(1))
      def _grp(g):
        off = coff + g * np.int32(16)
        ux0 = xs[pl.ds(off, 16)] * scale
        ux1 = xs[pl.ds(np.int32(PW) + off, 16)] * scale
        ux2 = xs[pl.ds(np.int32(2 * PW) + off, 16)] * scale
        i0 = ux0.astype(jnp.int32)
        i1 = ux1.astype(jnp.int32)
        i2 = ux2.astype(jnp.int32)
        d0 = ux0 - i0.astype(jnp.float32)
        d1 = ux1 - i1.astype(jnp.float32)
        d2 = ux2 - i2.astype(jnp.float32)
        a0, b0 = i0, i0 + 1
        a1, b1 = i1 * PI1, (i1 + 1) * PI1
        a2, b2 = i2 * PI2, (i2 + 1) * PI2
        w00, w10 = 1.0 - d0, d0
        w01, w11 = 1.0 - d1, d1
        w02, w12 = 1.0 - d2, d2
        gb = g * np.int32(16)
        for corner in range(8):
          t0 = b0 if (corner & 1) else a0
          t1 = b1 if (corner & 2) else a1
          t2 = b2 if (corner & 4) else a2
          e0 = ((t0 ^ t1 ^ t2) & MASK) * np.int32(2)
          w = ((w10 if (corner & 1) else w00)
               * (w11 if (corner & 2) else w01)
               * (w12 if (corner & 4) else w02))
          idx_v[pl.ds(np.int32(corner * C) + gb, 16)] = e0
          idx_v[pl.ds(np.int32(8 * C + corner * C) + gb, 16)] = (
              e0 + np.int32(1))
          wbuf[pl.ds(np.int32(corner * C) + gb, 16)] = w

      copies = [
          pltpu.async_copy(
              spm.at[idx_v.at[pl.ds(j * GATHER_W, GATHER_W)]],
              feat_v.at[pl.ds(j * GATHER_W, GATHER_W)],
              sem)
          for j in range(NGATHER)
      ]
      for cpy in copies:
        cpy.wait()

      @pl.loop(np.int32(0), np.int32(NG), step=np.int32(1))
      def _fma(g):
        gb = g * np.int32(16)
        acc0 = jnp.zeros((16,), jnp.float32)
        acc1 = jnp.zeros((16,), jnp.float32)
        for corner in range(8):
          w = wbuf[pl.ds(np.int32(corner * C) + gb, 16)]
          f0 = feat_v[pl.ds(np.int32(corner * C) + gb, 16)]
          f1 = feat_v[pl.ds(np.int32(8 * C + corner * C) + gb, 16)]
          acc0 = acc0 + w * f0
          acc1 = acc1 + w * f1
        prow = (iota + gb) * np.int32(2)
        plsc.store_scatter(obuf, [prow], acc0)
        plsc.store_scatter(obuf, [prow + np.int32(1)], acc1)

      pltpu.sync_copy(
          obuf,
          out_hbm.at[pl.ds(
              np.int32(l * B * F) + (pbase + coff) * np.int32(F), C * F)])

    plsc.subcore_barrier()


@jax.jit
def kernel(x, tables):
  with _jax_config.enable_x64(False):
    return _kernel_x32(x, tables)


def _kernel_x32(x, tables):
  xT = x.T.reshape(3 * B)
  tbl = tables.reshape(L * T * F)

  mesh = plsc.VectorSubcoreMesh(core_axis_name="c", subcore_axis_name="s")
  cp = pltpu.CompilerParams(needs_layout_passes=False,
                            use_tc_tiling_on_sc=False)
  enc = functools.partial(
      pl.kernel,
      compiler_params=cp,
      out_type=jax.ShapeDtypeStruct((L * B * F,), jnp.float32),
      mesh=mesh,
      scratch_types=[
          pltpu.VMEM((3 * PW,), jnp.float32),
          pltpu.VMEM((NIDX,), jnp.int32),
          pltpu.VMEM((NIDX,), jnp.float32),
          pltpu.VMEM((8 * C,), jnp.float32),
          pltpu.VMEM((C * F,), jnp.float32),
          pltpu.VMEM_SHARED((T * F,), jnp.float32),
          pltpu.SemaphoreType.DMA,
      ],
  )(_encoder_body)
  out = enc(xT, tbl)
  return out.reshape(L, B, F).transpose(1, 0, 2).reshape(B, 2 * L)


# speedup vs baseline: 1.0029x; 1.0029x over previous
import functools
import math

import jax
import jax.numpy as jnp
import numpy as np
from jax import lax
from jax._src import config as _jax_config
from jax.experimental import pallas as pl
from jax.experimental.pallas import tpu as pltpu
from jax.experimental.pallas import tpu_sc as plsc

N_MAX = 2048
N_MIN = 16
L = 16
T = 2 ** 19
F = 2
B = 131072
MASK = T - 1

PI1 = np.int32(-1640531535)
PI2 = np.int32(805459861)

NC = 2
NS = 16
NW = NC * NS
PW = B // NW
C = 1024
NG = C // 16
NCHUNK = PW // C
GATHER_W = 1024
NIDX = 16 * C
NGATHER = NIDX // GATHER_W
TSLICE = T * F // NS

_b = math.exp((math.log(N_MAX) - math.log(N_MIN)) / L)
SCALES = [np.float32(N_MIN * (_b ** i)) for i in range(L)]


def _encoder_body(xT_hbm, tbl_hbm, out_hbm,
                  xs, idx_v, feat_v, wbuf, obuf, spm, sem):
  sid = lax.axis_index("s")
  wid = sid * np.int32(NC) + lax.axis_index("c")
  pbase = wid * np.int32(PW)

  for d in range(3):
    pltpu.sync_copy(xT_hbm.at[pl.ds(np.int32(d * B) + pbase, PW)],
                    xs.at[pl.ds(d * PW, PW)])

  iota = lax.iota(jnp.int32, 16)

  for l in range(L):
    scale = SCALES[l]
    soff = sid * np.int32(TSLICE)
    pltpu.async_copy(
        tbl_hbm.at[pl.ds(np.int32(l * T * F) + soff, TSLICE)],
        spm.at[pl.ds(soff, TSLICE)], sem).wait()
    plsc.subcore_barrier()

    @pl.loop(np.int32(0), np.int32(NCHUNK), step=np.int32(1))
    def _chunk(chunk):
      coff = chunk * np.int32(C)

      @pl.loop(np.int32(0), np.int32(NG), step=np.int32(1))
      def _grp(g):
        off = coff + g * np.int32(16)
        ux0 = xs[pl.ds(off, 16)] * scale
        ux1 = xs[pl.ds(np.int32(PW) + off, 16)] * scale
        ux2 = xs[pl.ds(np.int32(2 * PW) + off, 16)] * scale
        i0 = ux0.astype(jnp.int32)
        i1 = ux1.astype(jnp.int32)
        i2 = ux2.astype(jnp.int32)
        d0 = ux0 - i0.astype(jnp.float32)
        d1 = ux1 - i1.astype(jnp.float32)
        d2 = ux2 - i2.astype(jnp.float32)
        a0, b0 = i0, i0 + 1
        a1, b1 = i1 * PI1, (i1 + 1) * PI1
        a2, b2 = i2 * PI2, (i2 + 1) * PI2
        w00, w10 = 1.0 - d0, d0
        w01, w11 = 1.0 - d1, d1
        w02, w12 = 1.0 - d2, d2
        gb = g * np.int32(16)
        for corner in range(8):
          t0 = b0 if (corner & 1) else a0
          t1 = b1 if (corner & 2) else a1
          t2 = b2 if (corner & 4) else a2
          e0 = ((t0 ^ t1 ^ t2) & MASK) * np.int32(2)
          w = ((w10 if (corner & 1) else w00)
               * (w11 if (corner & 2) else w01)
               * (w12 if (corner & 4) else w02))
          idx_v[pl.ds(np.int32(corner * C) + gb, 16)] = e0
          idx_v[pl.ds(np.int32(8 * C + corner * C) + gb, 16)] = (
              e0 + np.int32(1))
          wbuf[pl.ds(np.int32(corner * C) + gb, 16)] = w

      copies = [
          pltpu.async_copy(
              spm.at[idx_v.at[pl.ds(j * GATHER_W, GATHER_W)]],
              feat_v.at[pl.ds(j * GATHER_W, GATHER_W)],
              sem)
          for j in range(NGATHER)
      ]
      for cpy in copies:
        cpy.wait()

      @pl.loop(np.int32(0), np.int32(NG), step=np.int32(1))
      def _fma(g):
        gb = g * np.int32(16)
        acc0 = jnp.zeros((16,), jnp.float32)
        acc1 = jnp.zeros((16,), jnp.float32)
        for corner in range(8):
          w = wbuf[pl.ds(np.int32(corner * C) + gb, 16)]
          f0 = feat_v[pl.ds(np.int32(corner * C) + gb, 16)]
          f1 = feat_v[pl.ds(np.int32(8 * C + corner * C) + gb, 16)]
          acc0 = acc0 + w * f0
          acc1 = acc1 + w * f1
        prow = (iota + gb) * np.int32(2)
        plsc.store_scatter(obuf, [prow], acc0)
        plsc.store_scatter(obuf, [prow + np.int32(1)], acc1)

      pltpu.sync_copy(
          obuf,
          out_hbm.at[pl.ds(
              np.int32(l * B * F) + (pbase + coff) * np.int32(F), C * F)])

    plsc.subcore_barrier()


@jax.jit
def kernel(x, tables):
  with _jax_config.enable_x64(False):
    return _kernel_x32(x, tables)


def _kernel_x32(x, tables):
  xT = x.T.reshape(3 * B)
  tbl = tables.reshape(L * T * F)

  mesh = plsc.VectorSubcoreMesh(core_axis_name="c", subcore_axis_name="s")
  cp = pltpu.CompilerParams(needs_layout_passes=False,
                            use_tc_tiling_on_sc=False)
  enc = functools.partial(
      pl.kernel,
      compiler_params=cp,
      out_type=jax.ShapeDtypeStruct((L * B * F,), jnp.float32),
      mesh=mesh,
      scratch_types=[
          pltpu.VMEM((3 * PW,), jnp.float32),
          pltpu.VMEM((NIDX,), jnp.int32),
          pltpu.VMEM((NIDX,), jnp.float32),
          pltpu.VMEM((8 * C,), jnp.float32),
          pltpu.VMEM((C * F,), jnp.float32),
          pltpu.VMEM_SHARED((T * F,), jnp.float32),
          pltpu.SemaphoreType.DMA,
      ],
  )(_encoder_body)
  out = enc(xT, tbl)
  return out.reshape(L, B, F).transpose(1, 0, 2).reshape(B, 2 * L)
